# baseline (speedup 1.0000x reference)
import functools

import jax
import jax.numpy as jnp
from jax import lax
from jax.experimental import pallas as pl
from jax.experimental.pallas import tpu as pltpu
from jax.experimental.pallas import tpu_sc as plsc

T = 2048
D = 1024
H = 1024
NE = 8
K = 2
PAIRS = T * K
TILE = 128
NSLOTS = 5120
NTILES = NSLOTS // TILE


def _router_body(x_ref, wg_ref, ids_ref, sc_ref):
    x = x_ref[...]
    wg = wg_ref[...]
    logits = jnp.dot(x, wg, preferred_element_type=jnp.float32)
    lane = lax.broadcasted_iota(jnp.int32, logits.shape, 1)
    neg = jnp.float32(-1e30)
    logits = jnp.where(lane < NE, logits, neg)
    m1 = jnp.max(logits, axis=1, keepdims=True)
    i1 = jnp.min(jnp.where(logits == m1, lane, 128), axis=1, keepdims=True)
    l2 = jnp.where(lane == i1, neg, logits)
    m2 = jnp.max(l2, axis=1, keepdims=True)
    i2 = jnp.min(jnp.where(l2 == m2, lane, 128), axis=1, keepdims=True)
    z = jnp.sum(jnp.exp(logits - m1), axis=1, keepdims=True)
    s1 = 1.0 / z
    s2 = jnp.exp(m2 - m1) / z
    ids_ref[...] = jnp.concatenate([i1, i2], axis=1)
    sc_ref[...] = jnp.concatenate([s1, s2], axis=1)


def _router(x_flat, wg_pad):
    return pl.pallas_call(
        _router_body,
        out_shape=(
            jax.ShapeDtypeStruct((T, K), jnp.int32),
            jax.ShapeDtypeStruct((T, K), jnp.float32),
        ),
    )(x_flat, wg_pad)


def _dispatch_build_mock(ids):
    e = ids.reshape(-1)
    onehot = (e[:, None] == jnp.arange(NE)[None, :]).astype(jnp.int32)
    counts = jnp.sum(onehot, axis=0)
    padded = (counts + TILE - 1) // TILE * TILE
    offs = jnp.concatenate([jnp.zeros((1,), jnp.int32),
                            jnp.cumsum(padded)[:-1].astype(jnp.int32)])
    rank = jnp.cumsum(onehot, axis=0) - onehot
    pos = offs[e] + jnp.sum(rank * onehot, axis=1)
    tstart = offs // TILE
    tidx = jnp.arange(NTILES)
    tile_eid = jnp.sum((tidx[:, None] >= tstart[None, :]).astype(jnp.int32),
                       axis=1) - 1
    return pos.astype(jnp.int32), tile_eid.astype(jnp.int32)


def _gather_mock(x_flat, pos):
    tok = jnp.arange(PAIRS) // K
    return jnp.zeros((NSLOTS, D), jnp.float32).at[pos].set(x_flat[tok])


def _combine_mock(y, pos, scores):
    rows = y[pos.reshape(-1)].reshape(T, K, D)
    return jnp.sum(rows * scores[:, :, None], axis=1)


def _expert_body(eid_ref, xg_ref, w1_ref, w2_ref, wc_ref, y_ref):
    xg = xg_ref[...]
    w1 = w1_ref[0]
    w2 = w2_ref[0]
    wc = wc_ref[0]
    dn = (((1,), (1,)), ((), ()))
    h1 = lax.dot_general(xg, w1, dn, preferred_element_type=jnp.float32)
    h2 = lax.dot_general(xg, w2, dn, preferred_element_type=jnp.float32)
    h = (h1 * jax.nn.sigmoid(h1)) * h2
    y_ref[...] = lax.dot_general(h, wc, dn, preferred_element_type=jnp.float32)


def _expert_mlp(xg, w1, w2, wc, tile_eid):
    grid_spec = pltpu.PrefetchScalarGridSpec(
        num_scalar_prefetch=1,
        grid=(NTILES,),
        in_specs=[
            pl.BlockSpec((TILE, D), lambda i, eid: (i, 0)),
            pl.BlockSpec((1, H, D), lambda i, eid: (eid[i], 0, 0)),
            pl.BlockSpec((1, H, D), lambda i, eid: (eid[i], 0, 0)),
            pl.BlockSpec((1, D, H), lambda i, eid: (eid[i], 0, 0)),
        ],
        out_specs=pl.BlockSpec((TILE, D), lambda i, eid: (i, 0)),
    )
    return pl.pallas_call(
        _expert_body,
        grid_spec=grid_spec,
        out_shape=jax.ShapeDtypeStruct((NSLOTS, D), jnp.float32),
        compiler_params=pltpu.CompilerParams(
            dimension_semantics=("arbitrary",),
        ),
    )(tile_eid, xg, w1, w2, wc)


def kernel(x, W1, W2, Wc, Wg):
    b, s, d = x.shape
    x_flat = x.reshape(T, D)
    wg_pad = jnp.zeros((D, 128), jnp.float32).at[:, :NE].set(Wg.T)
    ids, scores = _router(x_flat, wg_pad)
    pos, tile_eid = _dispatch_build_mock(ids)
    xg = _gather_mock(x_flat, pos)
    y = _expert_mlp(xg, W1, W2, Wc, tile_eid)
    out = _combine_mock(y, pos, scores)
    return out.reshape(b, s, d)


# speedup vs baseline: 1.3673x; 1.3673x over previous
import functools

import jax
import jax.numpy as jnp
from jax import lax
from jax.experimental import pallas as pl
from jax.experimental.pallas import tpu as pltpu
from jax.experimental.pallas import tpu_sc as plsc

T = 2048
D = 1024
H = 1024
NE = 8
K = 2
PAIRS = T * K
TILE = 128
NSLOTS = 5120
NTILES = NSLOTS // TILE


NCHUNK = 32
TOK_PER_CHUNK = T // NCHUNK


def _router_body(x_ref, wg_ref, tri_ref, ids_ref, sc_ref, cnt_ref, aux_ref):
    x = x_ref[...]
    wg = wg_ref[...]
    logits = jnp.dot(x, wg, preferred_element_type=jnp.float32)
    lane = lax.broadcasted_iota(jnp.int32, logits.shape, 1)
    neg = jnp.float32(-1e30)
    logits = jnp.where(lane < NE, logits, neg)
    m1 = jnp.max(logits, axis=1, keepdims=True)
    i1 = jnp.min(jnp.where(logits == m1, lane, 128), axis=1, keepdims=True)
    l2 = jnp.where(lane == i1, neg, logits)
    m2 = jnp.max(l2, axis=1, keepdims=True)
    i2 = jnp.min(jnp.where(l2 == m2, lane, 128), axis=1, keepdims=True)
    z = jnp.sum(jnp.exp(logits - m1), axis=1, keepdims=True)
    s1 = 1.0 / z
    s2 = jnp.exp(m2 - m1) / z
    ids_ref[...] = jnp.concatenate([i1, i2], axis=1)
    sc_ref[...] = jnp.concatenate([s1, s2], axis=1)
    lane3 = lax.broadcasted_iota(jnp.int32, (NCHUNK, TOK_PER_CHUNK, 128), 2)
    i1r = i1.reshape(NCHUNK, TOK_PER_CHUNK, 1)
    i2r = i2.reshape(NCHUNK, TOK_PER_CHUNK, 1)
    hits = (lane3 == i1r).astype(jnp.int32) + (lane3 == i2r).astype(jnp.int32)
    cnts = jnp.sum(hits, axis=1)
    cnt_ref[...] = cnts
    totals = jnp.sum(cnts, axis=0, keepdims=True).astype(jnp.float32)
    padded = jnp.floor((totals + (TILE - 1)) / TILE) * TILE
    offs = jnp.dot(padded, tri_ref[...],
                   preferred_element_type=jnp.float32)
    offs_i = offs.astype(jnp.int32)
    lane2 = lax.broadcasted_iota(jnp.int32, (1, 128), 1)
    te = jnp.zeros((1, 128), jnp.int32) - 1
    for e in range(NE):
        tstart_e = offs_i[0, e] // TILE
        te = te + (lane2 >= tstart_e).astype(jnp.int32)
    aux_ref[...] = jnp.concatenate([offs_i, te], axis=0)


def _router(x_flat, wg_pad, tri):
    return pl.pallas_call(
        _router_body,
        out_shape=(
            jax.ShapeDtypeStruct((T, K), jnp.int32),
            jax.ShapeDtypeStruct((T, K), jnp.float32),
            jax.ShapeDtypeStruct((NCHUNK, 128), jnp.int32),
            jax.ShapeDtypeStruct((2, 128), jnp.int32),
        ),
    )(x_flat, wg_pad, tri)


_SC_MESH = plsc.VectorSubcoreMesh(core_axis_name="c", subcore_axis_name="s")


@functools.partial(
    pl.kernel,
    out_type=(
        jax.ShapeDtypeStruct((PAIRS,), jnp.int32),
        jax.ShapeDtypeStruct((NSLOTS, D), jnp.float32),
    ),
    mesh=_SC_MESH,
    scratch_types=[
        pltpu.VMEM((NCHUNK, 16), jnp.int32),
        pltpu.VMEM((16,), jnp.int32),
        pltpu.VMEM((128,), jnp.int32),
        pltpu.VMEM((4, 32), jnp.int32),
        pltpu.VMEM((4, 32), jnp.int32),
        pltpu.VMEM((32, D), jnp.float32),
        pltpu.SemaphoreType.DMA,
    ],
    compiler_params=pltpu.CompilerParams(needs_layout_passes=False),
)
def _dispatch_kernel(ids_hbm, cnts_hbm, offs_hbm, x_hbm, pos_hbm, xg_hbm,
                     cbuf, offbuf, idv, posbuf, tokidx, rows, sem):
    w = lax.axis_index("s") * 2 + lax.axis_index("c")
    pltpu.sync_copy(cnts_hbm, cbuf)
    pltpu.sync_copy(offs_hbm, offbuf)
    pltpu.sync_copy(ids_hbm.at[pl.ds(w * 128, 128)], idv)
    lane = lax.iota(jnp.int32, 16)
    zero = jnp.zeros((16,), jnp.int32)
    basev = offbuf[...]
    for t in range(NCHUNK):
        pred = jnp.where(t < w, 1, 0).astype(jnp.int32)
        basev = basev + cbuf[t, :] * pred
    for s4 in range(4):
        for h in range(2):
            vidx = s4 * 32 + h * 16
            v = idv[pl.ds(vidx, 16)]
            pos_v = zero
            hist = zero
            for e in range(NE):
                m = v == e
                inc = plsc.cumsum(jnp.where(m, 1, 0).astype(jnp.int32))
                pos_v = jnp.where(m, basev[e] + inc - 1, pos_v)
                pc = plsc.all_reduce_population_count(m)
                hist = jnp.where(lane == e, pc, hist)
            basev = basev + hist
            posbuf[s4, pl.ds(h * 16, 16)] = pos_v
            tokidx[s4, pl.ds(h * 16, 16)] = (w * 128 + vidx + lane) // 2
        pltpu.sync_copy(posbuf.at[s4],
                        pos_hbm.at[pl.ds(w * 128 + s4 * 32, 32)])
        pltpu.async_copy(x_hbm.at[tokidx.at[s4]], rows, sem).wait()
        pltpu.async_copy(rows, xg_hbm.at[posbuf.at[s4]], sem).wait()


@functools.partial(
    pl.kernel,
    out_type=jax.ShapeDtypeStruct((T, D), jnp.float32),
    mesh=_SC_MESH,
    scratch_types=[
        pltpu.VMEM((4, 32), jnp.int32),
        pltpu.VMEM((32,), jnp.float32),
        pltpu.VMEM((32, D), jnp.float32),
        pltpu.VMEM((16, D), jnp.float32),
        pltpu.SemaphoreType.DMA,
    ],
    compiler_params=pltpu.CompilerParams(needs_layout_passes=False),
)
def _combine_kernel(y_hbm, pos_hbm, sc_hbm, out_hbm,
                    posbuf, sbuf, yrows, obuf, sem):
    w = lax.axis_index("s") * 2 + lax.axis_index("c")
    for s4 in range(4):
        pltpu.sync_copy(pos_hbm.at[pl.ds(w * 128 + s4 * 32, 32)],
                        posbuf.at[s4])
        pltpu.sync_copy(sc_hbm.at[pl.ds(w * 128 + s4 * 32, 32)], sbuf)
        pltpu.async_copy(y_hbm.at[posbuf.at[s4]], yrows, sem).wait()
        sv0 = sbuf[pl.ds(0, 16)]
        sv1 = sbuf[pl.ds(16, 16)]
        for i in range(16):
            sv = sv0 if 2 * i < 16 else sv1
            sa = sv[(2 * i) % 16]
            sb = sv[(2 * i + 1) % 16]

            def body(vi, _, i=i, sa=sa, sb=sb):
                c = vi * 16
                a = yrows[2 * i, pl.ds(c, 16)]
                bv = yrows[2 * i + 1, pl.ds(c, 16)]
                obuf[i, pl.ds(c, 16)] = sa * a + sb * bv
                return 0

            lax.fori_loop(0, D // 16, body, 0)
        pltpu.sync_copy(obuf, out_hbm.at[pl.ds(w * 64 + s4 * 16, 16)])


def _expert_body(eid_ref, xg_ref, w1_ref, w2_ref, wc_ref, y_ref):
    xg = xg_ref[...]
    w1 = w1_ref[0]
    w2 = w2_ref[0]
    wc = wc_ref[0]
    dn = (((1,), (1,)), ((), ()))
    h1 = lax.dot_general(xg, w1, dn, preferred_element_type=jnp.float32)
    h2 = lax.dot_general(xg, w2, dn, preferred_element_type=jnp.float32)
    h = (h1 * jax.nn.sigmoid(h1)) * h2
    y_ref[...] = lax.dot_general(h, wc, dn, preferred_element_type=jnp.float32)


def _expert_mlp(xg, w1, w2, wc, tile_eid):
    grid_spec = pltpu.PrefetchScalarGridSpec(
        num_scalar_prefetch=1,
        grid=(NTILES,),
        in_specs=[
            pl.BlockSpec((TILE, D), lambda i, eid: (i, 0)),
            pl.BlockSpec((1, H, D), lambda i, eid: (eid[i], 0, 0)),
            pl.BlockSpec((1, H, D), lambda i, eid: (eid[i], 0, 0)),
            pl.BlockSpec((1, D, H), lambda i, eid: (eid[i], 0, 0)),
        ],
        out_specs=pl.BlockSpec((TILE, D), lambda i, eid: (i, 0)),
    )
    return pl.pallas_call(
        _expert_body,
        grid_spec=grid_spec,
        out_shape=jax.ShapeDtypeStruct((NSLOTS, D), jnp.float32),
        compiler_params=pltpu.CompilerParams(
            dimension_semantics=("arbitrary",),
        ),
    )(tile_eid, xg, w1, w2, wc)


def kernel(x, W1, W2, Wc, Wg):
    b, s, d = x.shape
    x_flat = x.reshape(T, D)
    wg_pad = jnp.zeros((D, 128), jnp.float32).at[:, :NE].set(Wg.T)
    tri = (jnp.arange(128)[:, None] < jnp.arange(128)[None, :]
           ).astype(jnp.float32)
    ids, scores, cnts, aux = _router(x_flat, wg_pad, tri)
    tile_eid = aux[1, :NTILES]
    pos, xg = _dispatch_kernel(ids.reshape(-1), cnts[:, :16], aux[0, :16],
                               x_flat)
    y = _expert_mlp(xg, W1, W2, Wc, tile_eid)
    out = _combine_kernel(y, pos, scores.reshape(-1))
    return out.reshape(b, s, d)


# speedup vs baseline: 1.7044x; 1.2465x over previous
import functools

import jax
import jax.numpy as jnp
from jax import lax
from jax.experimental import pallas as pl
from jax.experimental.pallas import tpu as pltpu
from jax.experimental.pallas import tpu_sc as plsc

T = 2048
D = 1024
H = 1024
NE = 8
K = 2
PAIRS = T * K
TILE = 256
NSLOTS = 6144
NTILES = NSLOTS // TILE


NCHUNK = 32
TOK_PER_CHUNK = T // NCHUNK


def _router_body(x_ref, wg_ref, tri_ref, ids_ref, sc_ref, cnt_ref, aux_ref):
    x = x_ref[...]
    wg = wg_ref[...]
    logits = jnp.dot(x, wg, preferred_element_type=jnp.float32)
    lane = lax.broadcasted_iota(jnp.int32, logits.shape, 1)
    neg = jnp.float32(-1e30)
    logits = jnp.where(lane < NE, logits, neg)
    m1 = jnp.max(logits, axis=1, keepdims=True)
    i1 = jnp.min(jnp.where(logits == m1, lane, 128), axis=1, keepdims=True)
    l2 = jnp.where(lane == i1, neg, logits)
    m2 = jnp.max(l2, axis=1, keepdims=True)
    i2 = jnp.min(jnp.where(l2 == m2, lane, 128), axis=1, keepdims=True)
    z = jnp.sum(jnp.exp(logits - m1), axis=1, keepdims=True)
    s1 = 1.0 / z
    s2 = jnp.exp(m2 - m1) / z
    ids_ref[...] = jnp.concatenate([i1, i2], axis=1)
    sc_ref[...] = jnp.concatenate([s1, s2], axis=1)
    lane3 = lax.broadcasted_iota(jnp.int32, (NCHUNK, TOK_PER_CHUNK, 128), 2)
    i1r = i1.reshape(NCHUNK, TOK_PER_CHUNK, 1)
    i2r = i2.reshape(NCHUNK, TOK_PER_CHUNK, 1)
    hits = (lane3 == i1r).astype(jnp.int32) + (lane3 == i2r).astype(jnp.int32)
    cnts = jnp.sum(hits, axis=1)
    cnt_ref[...] = cnts
    totals = jnp.sum(cnts, axis=0, keepdims=True).astype(jnp.float32)
    padded = jnp.floor((totals + (TILE - 1)) / TILE) * TILE
    offs = jnp.dot(padded, tri_ref[...],
                   preferred_element_type=jnp.float32)
    offs_i = offs.astype(jnp.int32)
    lane2 = lax.broadcasted_iota(jnp.int32, (1, 128), 1)
    te = jnp.zeros((1, 128), jnp.int32) - 1
    for e in range(NE):
        tstart_e = offs_i[0, e] // TILE
        te = te + (lane2 >= tstart_e).astype(jnp.int32)
    aux_ref[...] = jnp.concatenate([offs_i, te], axis=0)


def _router(x_flat, wg_pad, tri):
    return pl.pallas_call(
        _router_body,
        out_shape=(
            jax.ShapeDtypeStruct((T, K), jnp.int32),
            jax.ShapeDtypeStruct((T, K), jnp.float32),
            jax.ShapeDtypeStruct((NCHUNK, 128), jnp.int32),
            jax.ShapeDtypeStruct((2, 128), jnp.int32),
        ),
    )(x_flat, wg_pad, tri)


@functools.cache
def _dispatch_kernel_build():
    mesh = plsc.VectorSubcoreMesh(core_axis_name="c", subcore_axis_name="s")
    return pl.kernel(
        _dispatch_body,
        out_type=(
            jax.ShapeDtypeStruct((PAIRS,), jnp.int32),
            jax.ShapeDtypeStruct((NSLOTS, D), jnp.float32),
        ),
        mesh=mesh,
        scratch_types=[
            pltpu.VMEM((NCHUNK, 16), jnp.int32),
            pltpu.VMEM((16,), jnp.int32),
            pltpu.VMEM((128,), jnp.int32),
            pltpu.VMEM((4, 32), jnp.int32),
            pltpu.VMEM((4, 32), jnp.int32),
            pltpu.VMEM((32, D), jnp.float32),
            pltpu.SemaphoreType.DMA,
        ],
        compiler_params=pltpu.CompilerParams(needs_layout_passes=False),
    )


def _dispatch_body(ids_hbm, cnts_hbm, offs_hbm, x_hbm, pos_hbm, xg_hbm,
                   cbuf, offbuf, idv, posbuf, tokidx, rows, sem):
    w = lax.axis_index("s") * 2 + lax.axis_index("c")
    pltpu.sync_copy(cnts_hbm, cbuf)
    pltpu.sync_copy(offs_hbm, offbuf)
    pltpu.sync_copy(ids_hbm.at[pl.ds(w * 128, 128)], idv)
    lane = lax.iota(jnp.int32, 16)
    zero = jnp.zeros((16,), jnp.int32)
    basev = offbuf[...]
    for t in range(NCHUNK):
        pred = jnp.where(t < w, 1, 0).astype(jnp.int32)
        basev = basev + cbuf[t, :] * pred
    for s4 in range(4):
        for h in range(2):
            vidx = s4 * 32 + h * 16
            v = idv[pl.ds(vidx, 16)]
            pos_v = zero
            hist = zero
            for e in range(NE):
                m = v == e
                inc = plsc.cumsum(jnp.where(m, 1, 0).astype(jnp.int32))
                pos_v = jnp.where(m, basev[e] + inc - 1, pos_v)
                pc = plsc.all_reduce_population_count(m)
                hist = jnp.where(lane == e, pc, hist)
            basev = basev + hist
            posbuf[s4, pl.ds(h * 16, 16)] = pos_v
            tokidx[s4, pl.ds(h * 16, 16)] = (w * 128 + vidx + lane) // 2
        pltpu.sync_copy(posbuf.at[s4],
                        pos_hbm.at[pl.ds(w * 128 + s4 * 32, 32)])
        pltpu.async_copy(x_hbm.at[tokidx.at[s4]], rows, sem).wait()
        pltpu.async_copy(rows, xg_hbm.at[posbuf.at[s4]], sem).wait()


@functools.cache
def _combine_kernel_build():
    mesh = plsc.VectorSubcoreMesh(core_axis_name="c", subcore_axis_name="s")
    return pl.kernel(
        _combine_body,
        out_type=jax.ShapeDtypeStruct((T, D), jnp.float32),
        mesh=mesh,
        scratch_types=[
            pltpu.VMEM((4, 32), jnp.int32),
            pltpu.VMEM((32,), jnp.float32),
            pltpu.VMEM((32, D), jnp.float32),
            pltpu.VMEM((16, D), jnp.float32),
            pltpu.SemaphoreType.DMA,
        ],
        compiler_params=pltpu.CompilerParams(needs_layout_passes=False),
    )


def _combine_body(y_hbm, pos_hbm, sc_hbm, out_hbm,
                  posbuf, sbuf, yrows, obuf, sem):
    w = lax.axis_index("s") * 2 + lax.axis_index("c")
    for s4 in range(4):
        pltpu.sync_copy(pos_hbm.at[pl.ds(w * 128 + s4 * 32, 32)],
                        posbuf.at[s4])
        pltpu.sync_copy(sc_hbm.at[pl.ds(w * 128 + s4 * 32, 32)], sbuf)
        pltpu.async_copy(y_hbm.at[posbuf.at[s4]], yrows, sem).wait()
        sv0 = sbuf[pl.ds(0, 16)]
        sv1 = sbuf[pl.ds(16, 16)]
        for i in range(16):
            sv = sv0 if 2 * i < 16 else sv1
            sa = sv[(2 * i) % 16]
            sb = sv[(2 * i + 1) % 16]

            def body(vi, _, i=i, sa=sa, sb=sb):
                c = vi * 16
                a = yrows[2 * i, pl.ds(c, 16)]
                bv = yrows[2 * i + 1, pl.ds(c, 16)]
                obuf[i, pl.ds(c, 16)] = sa * a + sb * bv
                return 0

            lax.fori_loop(0, D // 16, body, 0)
        pltpu.sync_copy(obuf, out_hbm.at[pl.ds(w * 64 + s4 * 16, 16)])


def _expert_body(eid_ref, xg_ref, w1_ref, w2_ref, wc_ref, y_ref):
    xg = xg_ref[...]
    w1 = w1_ref[0]
    w2 = w2_ref[0]
    wc = wc_ref[0]
    dn = (((1,), (1,)), ((), ()))
    h1 = lax.dot_general(xg, w1, dn, preferred_element_type=jnp.float32)
    h2 = lax.dot_general(xg, w2, dn, preferred_element_type=jnp.float32)
    h = (h1 * jax.nn.sigmoid(h1)) * h2
    y_ref[...] = lax.dot_general(h, wc, dn, preferred_element_type=jnp.float32)


def _expert_mlp(xg, w1, w2, wc, tile_eid):
    grid_spec = pltpu.PrefetchScalarGridSpec(
        num_scalar_prefetch=1,
        grid=(NTILES,),
        in_specs=[
            pl.BlockSpec((TILE, D), lambda i, eid: (i, 0)),
            pl.BlockSpec((1, H, D), lambda i, eid: (eid[i], 0, 0)),
            pl.BlockSpec((1, H, D), lambda i, eid: (eid[i], 0, 0)),
            pl.BlockSpec((1, D, H), lambda i, eid: (eid[i], 0, 0)),
        ],
        out_specs=pl.BlockSpec((TILE, D), lambda i, eid: (i, 0)),
    )
    return pl.pallas_call(
        _expert_body,
        grid_spec=grid_spec,
        out_shape=jax.ShapeDtypeStruct((NSLOTS, D), jnp.float32),
        compiler_params=pltpu.CompilerParams(
            dimension_semantics=("arbitrary",),
        ),
    )(tile_eid, xg, w1, w2, wc)


def kernel(x, W1, W2, Wc, Wg):
    b, s, d = x.shape
    x_flat = x.reshape(T, D)
    wg_pad = jnp.zeros((D, 128), jnp.float32).at[:, :NE].set(Wg.T)
    tri = (jnp.arange(128)[:, None] < jnp.arange(128)[None, :]
           ).astype(jnp.float32)
    ids, scores, cnts, aux = _router(x_flat, wg_pad, tri)
    tile_eid = aux[1, :NTILES]
    pos, xg = _dispatch_kernel_build()(ids.reshape(-1), cnts[:, :16],
                                       aux[0, :16], x_flat)
    y = _expert_mlp(xg, W1, W2, Wc, tile_eid)
    out = _combine_kernel_build()(y, pos, scores.reshape(-1))
    return out.reshape(b, s, d)
